# baseline (speedup 1.0000x reference)
import functools

import jax
import jax.numpy as jnp
from jax import lax
from jax.experimental import pallas as pl
from jax.experimental.pallas import tpu as pltpu
from jax.experimental.pallas import tpu_sc as plsc

D = 1024
LANES = 16


def kernel(input_tensor, token_type_ids, token_type_table):
    b, s, e = input_tensor.shape
    n = b * s
    x2 = input_tensor.reshape(n, e)
    ids = token_type_ids.reshape(n).astype(jnp.int32)

    nw = 32
    rows_pw = n // nw
    chunk = 32
    n_chunks = rows_pw // chunk

    mesh = plsc.VectorSubcoreMesh(core_axis_name="c", subcore_axis_name="s")

    @functools.partial(
        pl.kernel,
        mesh=mesh,
        out_type=jax.ShapeDtypeStruct((n, e), jnp.float32),
        scratch_types=[
            pltpu.VMEM((chunk,), jnp.int32),
            pltpu.VMEM((chunk, e), jnp.float32),
            pltpu.VMEM((chunk, e), jnp.float32),
            pltpu.SemaphoreType.DMA,
        ],
    )
    def run(x_hbm, ids_hbm, tbl_hbm, out_hbm, ids_v, in_v, rows_v, sem):
        wid = lax.axis_index("s") * 2 + lax.axis_index("c")
        base = wid * rows_pw

        def chunk_body(ci, _):
            r0 = base + ci * chunk
            pltpu.sync_copy(ids_hbm.at[pl.ds(r0, chunk)], ids_v)
            gather = pltpu.async_copy(tbl_hbm.at[ids_v], rows_v, sem)
            pltpu.sync_copy(x_hbm.at[pl.ds(r0, chunk)], in_v)
            gather.wait()

            def row_body(r, _):
                def col_body(c, _):
                    sl = pl.ds(c * LANES, LANES)
                    in_v[r, sl] = in_v[r, sl] + rows_v[r, sl]
                    return 0

                return lax.fori_loop(0, e // LANES, col_body, 0)

            lax.fori_loop(0, chunk, row_body, 0)
            pltpu.sync_copy(in_v, out_hbm.at[pl.ds(r0, chunk)])
            return 0

        lax.fori_loop(0, n_chunks, chunk_body, 0)

    out = run(x2, ids, token_type_table)
    return out.reshape(b, s, e)
